# baseline (speedup 1.0000x reference)
import jax
import jax.numpy as jnp
from jax import lax
from jax.experimental import pallas as pl
from jax.experimental.pallas import tpu as pltpu
from jax.experimental.pallas import tpu_sc as plsc

N = 10000
E = 320000
NR = 10240
STRIPE = NR // 16
CHUNK = 128
IB = 8
EP = 327680
NB = 2
IN_CH = 128
HID_CH = 256
OUT_CH = 128
EPS = 1e-5


def _make_sc_agg(width, n_chunks, gather):
    mesh = plsc.VectorSubcoreMesh(core_axis_name="c", subcore_axis_name="s")
    n_groups = n_chunks // IB

    def body(*refs):
        if gather:
            (table, srcb, dstb, zeros, out,
             srcv, dstv, r0, r1, acc, sg0, sg1, ss0, ss1) = refs
            rows = (r0, r1)
            semg = (sg0, sg1)
            sems = (ss0, ss1)
        else:
            ones, dstb, zeros, out, dstv, ones_v, acc = refs
        c = lax.axis_index("c")
        s = lax.axis_index("s")
        pltpu.sync_copy(zeros, acc.at[pl.ds(s * STRIPE, STRIPE)])
        if not gather:
            pltpu.sync_copy(ones, ones_v)
        plsc.subcore_barrier()

        if gather:
            def group(g, carry):
                pltpu.sync_copy(srcb.at[c, s, pl.ds(g * IB, IB)], srcv)
                pltpu.sync_copy(dstb.at[c, s, pl.ds(g * IB, IB)], dstv)
                pltpu.async_copy(table.at[srcv.at[0]], rows[0], semg[0])

                def step(i, carry2):
                    j0 = 2 * i
                    j1 = j0 + 1
                    pltpu.async_copy(table.at[srcv.at[j1]], rows[1], semg[1])
                    pltpu.make_async_copy(table.at[srcv.at[j0]], rows[0],
                                          semg[0]).wait()
                    pltpu.make_async_copy(table.at[srcv.at[j1]], rows[1],
                                          semg[1]).wait()

                    @pl.when(j0 + 2 < IB)
                    def _():
                        pltpu.async_copy(table.at[srcv.at[j0 + 2]], rows[0],
                                         semg[0])

                    @pl.when(j1 + 2 < IB)
                    def _():
                        pltpu.async_copy(table.at[srcv.at[j1 + 2]], rows[1],
                                         semg[1])
                    return carry2

                lax.fori_loop(0, IB // 2, step, 0)
                return carry

            lax.fori_loop(0, n_groups, group, 0)
        else:
            def group(g, carry):
                pltpu.sync_copy(dstb.at[c, s, pl.ds(g * IB, IB)], dstv)

                def step(j, carry2):
                    pltpu.sync_copy(ones_v, acc.at[dstv.at[j]], add=True)
                    return carry2

                lax.fori_loop(0, IB, step, 0)
                return carry

            lax.fori_loop(0, n_groups, group, 0)

        plsc.subcore_barrier()
        pltpu.sync_copy(acc.at[pl.ds(s * STRIPE, STRIPE)],
                        out.at[c, pl.ds(s * STRIPE, STRIPE)])

    if gather:
        scratch = [
            pltpu.VMEM((IB, CHUNK), jnp.int32),
            pltpu.VMEM((IB, CHUNK), jnp.int32),
            pltpu.VMEM((CHUNK, width), jnp.float32),
            pltpu.VMEM((CHUNK, width), jnp.float32),
            pltpu.VMEM_SHARED((NR, width), jnp.float32),
            pltpu.SemaphoreType.DMA,
            pltpu.SemaphoreType.DMA,
            pltpu.SemaphoreType.DMA,
            pltpu.SemaphoreType.DMA,
        ]
    else:
        scratch = [
            pltpu.VMEM((IB, CHUNK), jnp.int32),
            pltpu.VMEM((CHUNK, width), jnp.float32),
            pltpu.VMEM_SHARED((NR, width), jnp.float32),
        ]
    return pl.kernel(
        body,
        out_type=jax.ShapeDtypeStruct((2, NR, width), jnp.float32),
        mesh=mesh,
        scratch_types=scratch,
    )


_N_CS = EP // (16 * CHUNK)
_N_ES = EP // (32 * CHUNK)

_sc_deg = _make_sc_agg(128, _N_ES, gather=False)
_sc_agg1 = _make_sc_agg(IN_CH, _N_CS, gather=True)
_sc_agg2 = _make_sc_agg(OUT_CH, _N_ES, gather=True)


def _tc_prep(x_ref, w1_ref, degp_ref, y_ref, dinv_ref):
    deg = degp_ref[0, :, 0:1] + degp_ref[1, :, 0:1] + 1.0
    dinv = lax.rsqrt(jnp.maximum(deg, 1.0))
    dinv_ref[...] = dinv
    xw = jnp.dot(x_ref[...], w1_ref[...], preferred_element_type=jnp.float32)
    y = xw * dinv[:N]
    y_ref[0, :N, :] = y[:, :IN_CH]
    y_ref[1, :N, :] = y[:, IN_CH:]


def _tc_mid(agg_ref, y1_ref, dinv_ref, b1_ref, g1_ref, be1_ref, w2_ref,
            y2_ref):
    dinv = dinv_ref[:N]
    hs = []
    for c in range(2):
        sl = slice(c * IN_CH, (c + 1) * IN_CH)
        t = (agg_ref[c, :N, :] + y1_ref[c, :N, :]) * dinv + b1_ref[:, sl]
        m = jnp.mean(t, axis=0, keepdims=True)
        v = jnp.mean(t * t, axis=0, keepdims=True) - m * m
        h = (t - m) * lax.rsqrt(v + EPS) * g1_ref[:, sl] + be1_ref[:, sl]
        hs.append(jnp.maximum(h, 0.0))
    y2 = (jnp.dot(hs[0], w2_ref[:IN_CH, :], preferred_element_type=jnp.float32)
          + jnp.dot(hs[1], w2_ref[IN_CH:, :],
                    preferred_element_type=jnp.float32))
    y2_ref[:N, :] = y2 * dinv


def _tc_fin(aggp_ref, y2_ref, dinv_ref, b2_ref, g2_ref, be2_ref, out_ref):
    dinv = dinv_ref[:N]
    t = ((aggp_ref[0, :N, :] + aggp_ref[1, :N, :] + y2_ref[:N, :]) * dinv
         + b2_ref[...])
    m = jnp.mean(t, axis=0, keepdims=True)
    v = jnp.mean(t * t, axis=0, keepdims=True) - m * m
    out_ref[...] = (t - m) * lax.rsqrt(v + EPS) * g2_ref[...] + be2_ref[...]


def kernel(x, edge_index, W1, b1, g1, be1, W2, b2, g2, be2):
    src = edge_index[0].astype(jnp.int32)
    dst = edge_index[1].astype(jnp.int32)
    pad = EP - E
    src_p = jnp.concatenate([src, jnp.zeros((pad,), jnp.int32)])
    dst_p = jnp.concatenate([dst, jnp.full((pad,), N, jnp.int32)])

    srcb_es = src_p.reshape(2, 16, -1, CHUNK)
    dstb_es = dst_p.reshape(2, 16, -1, CHUNK)
    src_cs = src_p.reshape(1, 16, -1, CHUNK)
    srcb_cs = jnp.concatenate([src_cs, src_cs + NR], axis=0)
    dstb_cs = jnp.broadcast_to(dst_p.reshape(1, 16, -1, CHUNK),
                               (2, 16, _N_CS, CHUNK))

    ones_t = jnp.ones((CHUNK, 128), jnp.float32)
    z128 = jnp.zeros((STRIPE, 128), jnp.float32)

    degp = _sc_deg(ones_t, dstb_es, z128)

    y1_tab, dinv = pl.pallas_call(
        _tc_prep,
        out_shape=(jax.ShapeDtypeStruct((2, NR, IN_CH), jnp.float32),
                   jax.ShapeDtypeStruct((NR, 1), jnp.float32)),
    )(x, W1, degp)

    agg1 = _sc_agg1(y1_tab.reshape(2 * NR, IN_CH), srcb_cs, dstb_cs, z128)

    y2_tab = pl.pallas_call(
        _tc_mid,
        out_shape=jax.ShapeDtypeStruct((NR, OUT_CH), jnp.float32),
    )(agg1, y1_tab, dinv, b1.reshape(1, -1), g1.reshape(1, -1),
      be1.reshape(1, -1), W2)

    agg2 = _sc_agg2(y2_tab, srcb_es, dstb_es, z128)

    out = pl.pallas_call(
        _tc_fin,
        out_shape=jax.ShapeDtypeStruct((N, OUT_CH), jnp.float32),
    )(agg2, y2_tab, dinv, b2.reshape(1, -1), g2.reshape(1, -1),
      be2.reshape(1, -1))
    return out


# speedup vs baseline: 1.1873x; 1.1873x over previous
import jax
import jax.numpy as jnp
from jax import lax
from jax.experimental import pallas as pl
from jax.experimental.pallas import tpu as pltpu
from jax.experimental.pallas import tpu_sc as plsc

N = 10000
E = 320000
NR = 10240
STRIPE = NR // 16
CHUNK = 128
IB = 8
EP = 327680
NB = 2
IN_CH = 128
HID_CH = 256
OUT_CH = 128
EPS = 1e-5


def _make_sc_agg(width, n_chunks, gather):
    mesh = plsc.VectorSubcoreMesh(core_axis_name="c", subcore_axis_name="s")
    n_groups = n_chunks // IB

    def body(*refs):
        if gather:
            (table, srcb, dstb, zeros, out, srcv, dstv, rows, acc) = refs
        else:
            ones, dstb, zeros, out, dstv, ones_v, acc = refs
        c = lax.axis_index("c")
        s = lax.axis_index("s")
        if gather:
            pltpu.sync_copy(zeros, acc.at[0, pl.ds(s * STRIPE, STRIPE)])
        else:
            pltpu.sync_copy(zeros, acc.at[pl.ds(s * STRIPE, STRIPE)])
        if not gather:
            pltpu.sync_copy(ones, ones_v)
        plsc.subcore_barrier()

        if gather:
            def group(g, carry):
                pltpu.sync_copy(srcb.at[c, s, pl.ds(g * IB, IB)], srcv)
                pltpu.sync_copy(dstb.at[c, s, pl.ds(g * IB, IB)], dstv)

                def step(j, carry2):
                    pltpu.sync_copy(table.at[srcv.at[j]], rows)
                    pltpu.sync_copy(rows, acc.at[dstv.at[j]], add=True)
                    return carry2

                lax.fori_loop(0, IB, step, 0)
                return carry

            lax.fori_loop(0, n_groups, group, 0)
        else:
            def group(g, carry):
                pltpu.sync_copy(dstb.at[c, s, pl.ds(g * IB, IB)], dstv)

                def step(j, carry2):
                    pltpu.sync_copy(ones_v, acc.at[dstv.at[j]], add=True)
                    return carry2

                lax.fori_loop(0, IB, step, 0)
                return carry

            lax.fori_loop(0, n_groups, group, 0)

        plsc.subcore_barrier()
        if gather:
            pltpu.sync_copy(acc.at[0, pl.ds(s * STRIPE, STRIPE)],
                            out.at[c, pl.ds(s * STRIPE, STRIPE)])
        else:
            pltpu.sync_copy(acc.at[pl.ds(s * STRIPE, STRIPE)],
                            out.at[c, pl.ds(s * STRIPE, STRIPE)])

    if gather:
        scratch = [
            pltpu.VMEM((IB, 1, 2 * CHUNK), jnp.int32),
            pltpu.VMEM((IB, 1, 2 * CHUNK), jnp.int32),
            pltpu.VMEM((1, 2 * CHUNK, width), jnp.float32),
            pltpu.VMEM_SHARED((1, NR, width), jnp.float32),
        ]
    else:
        scratch = [
            pltpu.VMEM((IB, CHUNK), jnp.int32),
            pltpu.VMEM((CHUNK, width), jnp.float32),
            pltpu.VMEM_SHARED((NR, width), jnp.float32),
        ]
    return pl.kernel(
        body,
        out_type=jax.ShapeDtypeStruct((2, NR, width), jnp.float32),
        mesh=mesh,
        scratch_types=scratch,
    )


_N_CS = EP // (16 * 2 * CHUNK)
_N_ES = EP // (32 * 2 * CHUNK)
_N_DEG = EP // (32 * CHUNK)

_sc_deg = _make_sc_agg(128, _N_DEG, gather=False)
_sc_agg1 = _make_sc_agg(IN_CH, _N_CS, gather=True)
_sc_agg2 = _make_sc_agg(OUT_CH, _N_ES, gather=True)


def _tc_prep(x_ref, w1_ref, degp_ref, y_ref, dinv_ref):
    deg = degp_ref[0, :, 0:1] + degp_ref[1, :, 0:1] + 1.0
    dinv = lax.rsqrt(jnp.maximum(deg, 1.0))
    dinv_ref[...] = dinv
    xw = jnp.dot(x_ref[...], w1_ref[...], preferred_element_type=jnp.float32)
    y = xw * dinv[:N]
    y_ref[0, :N, :] = y[:, :IN_CH]
    y_ref[1, :N, :] = y[:, IN_CH:]


def _tc_mid(agg_ref, y1_ref, dinv_ref, b1_ref, g1_ref, be1_ref, w2_ref,
            y2_ref):
    dinv = dinv_ref[:N]
    hs = []
    for c in range(2):
        sl = slice(c * IN_CH, (c + 1) * IN_CH)
        t = (agg_ref[c, :N, :] + y1_ref[c, :N, :]) * dinv + b1_ref[:, sl]
        m = jnp.mean(t, axis=0, keepdims=True)
        v = jnp.mean(t * t, axis=0, keepdims=True) - m * m
        h = (t - m) * lax.rsqrt(v + EPS) * g1_ref[:, sl] + be1_ref[:, sl]
        hs.append(jnp.maximum(h, 0.0))
    y2 = (jnp.dot(hs[0], w2_ref[:IN_CH, :], preferred_element_type=jnp.float32)
          + jnp.dot(hs[1], w2_ref[IN_CH:, :],
                    preferred_element_type=jnp.float32))
    y2_ref[:N, :] = y2 * dinv


def _tc_fin(aggp_ref, y2_ref, dinv_ref, b2_ref, g2_ref, be2_ref, out_ref):
    dinv = dinv_ref[:N]
    t = ((aggp_ref[0, :N, :] + aggp_ref[1, :N, :] + y2_ref[:N, :]) * dinv
         + b2_ref[...])
    m = jnp.mean(t, axis=0, keepdims=True)
    v = jnp.mean(t * t, axis=0, keepdims=True) - m * m
    out_ref[...] = (t - m) * lax.rsqrt(v + EPS) * g2_ref[...] + be2_ref[...]


def kernel(x, edge_index, W1, b1, g1, be1, W2, b2, g2, be2):
    src = edge_index[0].astype(jnp.int32)
    dst = edge_index[1].astype(jnp.int32)
    pad = EP - E
    src_p = jnp.concatenate([src, jnp.zeros((pad,), jnp.int32)])
    dst_p = jnp.concatenate([dst, jnp.full((pad,), N, jnp.int32)])

    srcb_es = src_p.reshape(2, 16, -1, 1, 2 * CHUNK)
    dstb_es = dst_p.reshape(2, 16, -1, 1, 2 * CHUNK)
    dstb_deg = dst_p.reshape(2, 16, -1, CHUNK)
    src_cs = src_p.reshape(1, 16, -1, 1, 2 * CHUNK)
    srcb_cs = jnp.concatenate([src_cs, src_cs + NR], axis=0)
    dstb_cs = jnp.broadcast_to(dst_p.reshape(1, 16, -1, 1, 2 * CHUNK),
                               (2, 16, _N_CS, 1, 2 * CHUNK))

    ones_t = jnp.ones((CHUNK, 128), jnp.float32)
    z128 = jnp.zeros((STRIPE, 128), jnp.float32)

    degp = _sc_deg(ones_t, dstb_deg, z128)

    y1_tab, dinv = pl.pallas_call(
        _tc_prep,
        out_shape=(jax.ShapeDtypeStruct((2, NR, IN_CH), jnp.float32),
                   jax.ShapeDtypeStruct((NR, 1), jnp.float32)),
    )(x, W1, degp)

    agg1 = _sc_agg1(y1_tab.reshape(1, 2 * NR, IN_CH), srcb_cs, dstb_cs,
                    z128)

    y2_tab = pl.pallas_call(
        _tc_mid,
        out_shape=jax.ShapeDtypeStruct((NR, OUT_CH), jnp.float32),
    )(agg1, y1_tab, dinv, b1.reshape(1, -1), g1.reshape(1, -1),
      be1.reshape(1, -1), W2)

    agg2 = _sc_agg2(y2_tab.reshape(1, NR, OUT_CH), srcb_es, dstb_es, z128)

    out = pl.pallas_call(
        _tc_fin,
        out_shape=jax.ShapeDtypeStruct((N, OUT_CH), jnp.float32),
    )(agg2, y2_tab, dinv, b2.reshape(1, -1), g2.reshape(1, -1),
      be2.reshape(1, -1))
    return out


# speedup vs baseline: 1.3462x; 1.1338x over previous
import jax
import jax.numpy as jnp
from jax import lax
from jax.experimental import pallas as pl
from jax.experimental.pallas import tpu as pltpu
from jax.experimental.pallas import tpu_sc as plsc

N = 10000
E = 320000
NR = 10240
STRIPE = NR // 16
CHUNK = 128
SCH = 160
IB = 8
EP = 327680
NB = 2
IN_CH = 128
HID_CH = 256
OUT_CH = 128
EPS = 1e-5


def _make_sc_agg(width, n_chunks, gather):
    mesh = plsc.VectorSubcoreMesh(core_axis_name="c", subcore_axis_name="s")
    n_groups = n_chunks // IB

    def body(*refs):
        if gather:
            (table, srcb, dstb, zeros, out,
             srcv, dstv, r0, r1, acc, sg0, sg1) = refs
            rows = (r0, r1)
            semg = (sg0, sg1)
        else:
            ones, dstb, zeros, out, dstv, ones_v, acc = refs
        c = lax.axis_index("c")
        s = lax.axis_index("s")
        if gather:
            pltpu.sync_copy(zeros, acc.at[0, pl.ds(s * STRIPE, STRIPE)])
        else:
            pltpu.sync_copy(zeros, acc.at[pl.ds(s * STRIPE, STRIPE)])
        if not gather:
            pltpu.sync_copy(ones, ones_v)
        plsc.subcore_barrier()

        if gather:
            def group(g, carry):
                pltpu.sync_copy(srcb.at[c, s, pl.ds(g * IB, IB)], srcv)
                pltpu.sync_copy(dstb.at[c, s, pl.ds(g * IB, IB)], dstv)
                pltpu.async_copy(table.at[srcv.at[0]], rows[0], semg[0])

                def step(i, carry2):
                    j0 = 2 * i
                    j1 = j0 + 1
                    pltpu.async_copy(table.at[srcv.at[j1]], rows[1], semg[1])
                    pltpu.make_async_copy(table.at[srcv.at[j0]], rows[0],
                                          semg[0]).wait()
                    pltpu.sync_copy(rows[0], acc.at[dstv.at[j0]], add=True)

                    @pl.when(j0 + 2 < IB)
                    def _():
                        pltpu.async_copy(table.at[srcv.at[j0 + 2]], rows[0],
                                         semg[0])

                    pltpu.make_async_copy(table.at[srcv.at[j1]], rows[1],
                                          semg[1]).wait()
                    pltpu.sync_copy(rows[1], acc.at[dstv.at[j1]], add=True)
                    return carry2

                lax.fori_loop(0, IB // 2, step, 0)
                return carry

            lax.fori_loop(0, n_groups, group, 0)
        else:
            def group(g, carry):
                pltpu.sync_copy(dstb.at[c, s, pl.ds(g * IB, IB)], dstv)

                def step(j, carry2):
                    pltpu.sync_copy(ones_v, acc.at[dstv.at[j]], add=True)
                    return carry2

                lax.fori_loop(0, IB, step, 0)
                return carry

            lax.fori_loop(0, n_groups, group, 0)

        plsc.subcore_barrier()
        if gather:
            pltpu.sync_copy(acc.at[0, pl.ds(s * STRIPE, STRIPE)],
                            out.at[c, pl.ds(s * STRIPE, STRIPE)])
        else:
            pltpu.sync_copy(acc.at[pl.ds(s * STRIPE, STRIPE)],
                            out.at[c, pl.ds(s * STRIPE, STRIPE)])

    if gather:
        scratch = [
            pltpu.VMEM((IB, 1, SCH), jnp.int32),
            pltpu.VMEM((IB, 1, SCH), jnp.int32),
            pltpu.VMEM((1, SCH, width), jnp.float32),
            pltpu.VMEM((1, SCH, width), jnp.float32),
            pltpu.VMEM_SHARED((1, NR, width), jnp.float32),
            pltpu.SemaphoreType.DMA,
            pltpu.SemaphoreType.DMA,
        ]
    else:
        scratch = [
            pltpu.VMEM((IB, CHUNK), jnp.int32),
            pltpu.VMEM((CHUNK, width), jnp.float32),
            pltpu.VMEM_SHARED((NR, width), jnp.float32),
        ]
    return pl.kernel(
        body,
        out_type=jax.ShapeDtypeStruct((2, NR, width), jnp.float32),
        mesh=mesh,
        scratch_types=scratch,
    )


_N_CS = EP // (16 * SCH)
_N_ES = EP // (32 * SCH)
_N_DEG = EP // (32 * CHUNK)

_sc_deg = _make_sc_agg(128, _N_DEG, gather=False)
_sc_agg1 = _make_sc_agg(IN_CH, _N_CS, gather=True)
_sc_agg2 = _make_sc_agg(OUT_CH, _N_ES, gather=True)


def _tc_prep(x_ref, w1_ref, degp_ref, y_ref, dinv_ref):
    deg = degp_ref[0, :, 0:1] + degp_ref[1, :, 0:1] + 1.0
    dinv = lax.rsqrt(jnp.maximum(deg, 1.0))
    dinv_ref[...] = dinv
    xw = jnp.dot(x_ref[...], w1_ref[...], preferred_element_type=jnp.float32)
    y = xw * dinv[:N]
    y_ref[0, :N, :] = y[:, :IN_CH]
    y_ref[1, :N, :] = y[:, IN_CH:]


def _tc_mid(agg_ref, y1_ref, dinv_ref, b1_ref, g1_ref, be1_ref, w2_ref,
            y2_ref):
    dinv = dinv_ref[:N]
    hs = []
    for c in range(2):
        sl = slice(c * IN_CH, (c + 1) * IN_CH)
        t = (agg_ref[c, :N, :] + y1_ref[c, :N, :]) * dinv + b1_ref[:, sl]
        m = jnp.mean(t, axis=0, keepdims=True)
        v = jnp.mean(t * t, axis=0, keepdims=True) - m * m
        h = (t - m) * lax.rsqrt(v + EPS) * g1_ref[:, sl] + be1_ref[:, sl]
        hs.append(jnp.maximum(h, 0.0))
    y2 = (jnp.dot(hs[0], w2_ref[:IN_CH, :], preferred_element_type=jnp.float32)
          + jnp.dot(hs[1], w2_ref[IN_CH:, :],
                    preferred_element_type=jnp.float32))
    y2_ref[:N, :] = y2 * dinv


def _tc_fin(aggp_ref, y2_ref, dinv_ref, b2_ref, g2_ref, be2_ref, out_ref):
    dinv = dinv_ref[:N]
    t = ((aggp_ref[0, :N, :] + aggp_ref[1, :N, :] + y2_ref[:N, :]) * dinv
         + b2_ref[...])
    m = jnp.mean(t, axis=0, keepdims=True)
    v = jnp.mean(t * t, axis=0, keepdims=True) - m * m
    out_ref[...] = (t - m) * lax.rsqrt(v + EPS) * g2_ref[...] + be2_ref[...]


def kernel(x, edge_index, W1, b1, g1, be1, W2, b2, g2, be2):
    src = edge_index[0].astype(jnp.int32)
    dst = edge_index[1].astype(jnp.int32)
    pad = EP - E
    src_p = jnp.concatenate([src, jnp.zeros((pad,), jnp.int32)])
    dst_p = jnp.concatenate([dst, jnp.full((pad,), N, jnp.int32)])

    srcb_es = src_p.reshape(2, 16, -1, 1, SCH)
    dstb_es = dst_p.reshape(2, 16, -1, 1, SCH)
    dstb_deg = dst_p.reshape(2, 16, -1, CHUNK)
    src_cs = src_p.reshape(1, 16, -1, 1, SCH)
    srcb_cs = jnp.concatenate([src_cs, src_cs + NR], axis=0)
    dstb_cs = jnp.broadcast_to(dst_p.reshape(1, 16, -1, 1, SCH),
                               (2, 16, _N_CS, 1, SCH))

    ones_t = jnp.ones((CHUNK, 128), jnp.float32)
    z128 = jnp.zeros((STRIPE, 128), jnp.float32)

    degp = _sc_deg(ones_t, dstb_deg, z128)

    y1_tab, dinv = pl.pallas_call(
        _tc_prep,
        out_shape=(jax.ShapeDtypeStruct((2, NR, IN_CH), jnp.float32),
                   jax.ShapeDtypeStruct((NR, 1), jnp.float32)),
    )(x, W1, degp)

    agg1 = _sc_agg1(y1_tab.reshape(1, 2 * NR, IN_CH), srcb_cs, dstb_cs,
                    z128)

    y2_tab = pl.pallas_call(
        _tc_mid,
        out_shape=jax.ShapeDtypeStruct((NR, OUT_CH), jnp.float32),
    )(agg1, y1_tab, dinv, b1.reshape(1, -1), g1.reshape(1, -1),
      be1.reshape(1, -1), W2)

    agg2 = _sc_agg2(y2_tab.reshape(1, NR, OUT_CH), srcb_es, dstb_es, z128)

    out = pl.pallas_call(
        _tc_fin,
        out_shape=jax.ShapeDtypeStruct((N, OUT_CH), jnp.float32),
    )(agg2, y2_tab, dinv, b2.reshape(1, -1), g2.reshape(1, -1),
      be2.reshape(1, -1))
    return out


# speedup vs baseline: 3.2668x; 2.4266x over previous
import jax
import jax.numpy as jnp
from jax import lax
from jax.experimental import pallas as pl
from jax.experimental.pallas import tpu as pltpu
from jax.experimental.pallas import tpu_sc as plsc

N = 10000
E = 320000
NR = 10240
STRIPE = NR // 16
CHUNK = 128
SCH = 160
IB = 8
EP = 327680
NB = 2
IN_CH = 128
HID_CH = 256
OUT_CH = 128
EPS = 1e-5


def _make_sc_agg(width, n_chunks, gather):
    mesh = plsc.VectorSubcoreMesh(core_axis_name="c", subcore_axis_name="s")
    n_groups = n_chunks // IB

    def body(*refs):
        if gather:
            (table, srcb, dstb, zeros, out,
             srcv, dstv, r0, r1, acc, sg0, sg1) = refs
            rows = (r0, r1)
            semg = (sg0, sg1)
        else:
            ones, dstb, zeros, out, dstv, ones_v, acc = refs
        c = lax.axis_index("c")
        s = lax.axis_index("s")
        if gather:
            pltpu.sync_copy(zeros, acc.at[0, pl.ds(s * STRIPE, STRIPE)])
        else:
            pltpu.sync_copy(zeros, acc.at[pl.ds(s * STRIPE, STRIPE)])
        if not gather:
            pltpu.sync_copy(ones, ones_v)
        plsc.subcore_barrier()

        if gather:
            def group(g, carry):
                pltpu.sync_copy(srcb.at[c, s, pl.ds(g * IB, IB)], srcv)
                pltpu.sync_copy(dstb.at[c, s, pl.ds(g * IB, IB)], dstv)
                pltpu.async_copy(table.at[srcv.at[0]], rows[0], semg[0])

                def step(i, carry2):
                    j0 = 2 * i
                    j1 = j0 + 1
                    pltpu.async_copy(table.at[srcv.at[j1]], rows[1], semg[1])
                    pltpu.make_async_copy(table.at[srcv.at[j0]], rows[0],
                                          semg[0]).wait()
                    pltpu.sync_copy(rows[0], acc.at[dstv.at[j0]], add=True)

                    @pl.when(j0 + 2 < IB)
                    def _():
                        pltpu.async_copy(table.at[srcv.at[j0 + 2]], rows[0],
                                         semg[0])

                    pltpu.make_async_copy(table.at[srcv.at[j1]], rows[1],
                                          semg[1]).wait()
                    pltpu.sync_copy(rows[1], acc.at[dstv.at[j1]], add=True)
                    return carry2

                lax.fori_loop(0, IB // 2, step, 0)
                return carry

            lax.fori_loop(0, n_groups, group, 0)
        else:
            def group(g, carry):
                pltpu.sync_copy(dstb.at[c, s, pl.ds(g * IB, IB)], dstv)

                def step(j, carry2):
                    pltpu.sync_copy(ones_v, acc.at[dstv.at[j]], add=True)
                    return carry2

                lax.fori_loop(0, IB, step, 0)
                return carry

            lax.fori_loop(0, n_groups, group, 0)

        plsc.subcore_barrier()
        if gather:
            pltpu.sync_copy(acc.at[0, pl.ds(s * STRIPE, STRIPE)],
                            out.at[c, pl.ds(s * STRIPE, STRIPE)])
        else:
            pltpu.sync_copy(acc.at[pl.ds(s * STRIPE, STRIPE)],
                            out.at[c, pl.ds(s * STRIPE, STRIPE)])

    if gather:
        scratch = [
            pltpu.VMEM((IB, 1, SCH), jnp.int32),
            pltpu.VMEM((IB, 1, SCH), jnp.int32),
            pltpu.VMEM((1, SCH, width), jnp.float32),
            pltpu.VMEM((1, SCH, width), jnp.float32),
            pltpu.VMEM_SHARED((1, NR, width), jnp.float32),
            pltpu.SemaphoreType.DMA,
            pltpu.SemaphoreType.DMA,
        ]
    else:
        scratch = [
            pltpu.VMEM((IB, CHUNK), jnp.int32),
            pltpu.VMEM((CHUNK, width), jnp.float32),
            pltpu.VMEM_SHARED((NR, width), jnp.float32),
        ]
    return pl.kernel(
        body,
        out_type=jax.ShapeDtypeStruct((2, NR, width), jnp.float32),
        mesh=mesh,
        scratch_types=scratch,
    )


_N_CS = EP // (16 * SCH)
_N_ES = EP // (32 * SCH)
_N_DEG = EP // (32 * CHUNK)

_sc_deg = _make_sc_agg(128, _N_DEG, gather=False)
_sc_agg1 = _make_sc_agg(IN_CH, _N_CS, gather=True)
_sc_agg2 = _make_sc_agg(OUT_CH, _N_ES, gather=True)


def _tc_prep(x_ref, w1_ref, degp_ref, y_ref, dinv_ref):
    deg = degp_ref[0, :, 0:1] + degp_ref[1, :, 0:1] + 1.0
    dinv = lax.rsqrt(jnp.maximum(deg, 1.0))
    dinv_ref[...] = dinv
    xw = jnp.dot(x_ref[...], w1_ref[...], preferred_element_type=jnp.float32)
    y = xw * dinv[:N]
    y_ref[0, :N, :] = y[:, :IN_CH]
    y_ref[1, :N, :] = y[:, IN_CH:]


def _tc_mid(agg_ref, y1_ref, dinv_ref, b1_ref, g1_ref, be1_ref, w2_ref,
            y2_ref):
    dinv = dinv_ref[:N]
    hs = []
    for c in range(2):
        sl = slice(c * IN_CH, (c + 1) * IN_CH)
        t = (agg_ref[c, :N, :] + y1_ref[c, :N, :]) * dinv + b1_ref[:, sl]
        m = jnp.mean(t, axis=0, keepdims=True)
        v = jnp.mean(t * t, axis=0, keepdims=True) - m * m
        h = (t - m) * lax.rsqrt(v + EPS) * g1_ref[:, sl] + be1_ref[:, sl]
        hs.append(jnp.maximum(h, 0.0))
    y2 = (jnp.dot(hs[0], w2_ref[:IN_CH, :], preferred_element_type=jnp.float32)
          + jnp.dot(hs[1], w2_ref[IN_CH:, :],
                    preferred_element_type=jnp.float32))
    y2_ref[:N, :] = y2 * dinv


def _tc_fin(aggp_ref, y2_ref, dinv_ref, b2_ref, g2_ref, be2_ref, out_ref):
    dinv = dinv_ref[:N]
    t = ((aggp_ref[0, :N, :] + aggp_ref[1, :N, :] + y2_ref[:N, :]) * dinv
         + b2_ref[...])
    m = jnp.mean(t, axis=0, keepdims=True)
    v = jnp.mean(t * t, axis=0, keepdims=True) - m * m
    out_ref[...] = (t - m) * lax.rsqrt(v + EPS) * g2_ref[...] + be2_ref[...]


def kernel(x, edge_index, W1, b1, g1, be1, W2, b2, g2, be2):
    src = edge_index[0].astype(jnp.int32)
    dst = edge_index[1].astype(jnp.int32)
    pad = EP - E
    pad_iota = jnp.arange(pad, dtype=jnp.int32)
    src_p = jnp.concatenate([src, pad_iota % N])
    dst_p = jnp.concatenate([dst, N + pad_iota % (NR - N)])

    srcb_es = src_p.reshape(2, 16, -1, 1, SCH)
    dstb_es = dst_p.reshape(2, 16, -1, 1, SCH)
    dstb_deg = dst_p.reshape(2, 16, -1, CHUNK)
    src_cs = src_p.reshape(1, 16, -1, 1, SCH)
    srcb_cs = jnp.concatenate([src_cs, src_cs + NR], axis=0)
    dstb_cs = jnp.broadcast_to(dst_p.reshape(1, 16, -1, 1, SCH),
                               (2, 16, _N_CS, 1, SCH))

    ones_t = jnp.ones((CHUNK, 128), jnp.float32)
    z128 = jnp.zeros((STRIPE, 128), jnp.float32)

    degp = _sc_deg(ones_t, dstb_deg, z128)

    y1_tab, dinv = pl.pallas_call(
        _tc_prep,
        out_shape=(jax.ShapeDtypeStruct((2, NR, IN_CH), jnp.float32),
                   jax.ShapeDtypeStruct((NR, 1), jnp.float32)),
    )(x, W1, degp)

    agg1 = _sc_agg1(y1_tab.reshape(1, 2 * NR, IN_CH), srcb_cs, dstb_cs,
                    z128)

    y2_tab = pl.pallas_call(
        _tc_mid,
        out_shape=jax.ShapeDtypeStruct((NR, OUT_CH), jnp.float32),
    )(agg1, y1_tab, dinv, b1.reshape(1, -1), g1.reshape(1, -1),
      be1.reshape(1, -1), W2)

    agg2 = _sc_agg2(y2_tab.reshape(1, NR, OUT_CH), srcb_es, dstb_es, z128)

    out = pl.pallas_call(
        _tc_fin,
        out_shape=jax.ShapeDtypeStruct((N, OUT_CH), jnp.float32),
    )(agg2, y2_tab, dinv, b2.reshape(1, -1), g2.reshape(1, -1),
      be2.reshape(1, -1))
    return out
